# Initial kernel scaffold; baseline (speedup 1.0000x reference)
import jax
import jax.numpy as jnp
from jax import lax
from jax.experimental import pallas as pl
from jax.experimental.pallas import tpu as pltpu
from jax.experimental.pallas import tpu_sc as plsc

N_NODES = 10000
D = 128
N_CORES = 2
N_SUBCORES = 16
N_WORKERS = N_CORES * N_SUBCORES
EDGES_PER_TILE = 10000
CHUNK = 125
N_CHUNKS = EDGES_PER_TILE // CHUNK
ROWS_PER_TILE = N_NODES // N_SUBCORES


def _sc_body(x_hbm, src_hbm, dst_hbm, out_hbm,
             sidx, didx, rows0, rows1, zbuf, acc, sem0, sem1):
    c = lax.axis_index("c")
    s = lax.axis_index("s")
    wid = c * N_SUBCORES + s

    pltpu.sync_copy(src_hbm.at[wid], sidx)
    pltpu.sync_copy(dst_hbm.at[wid], didx)

    zero = jnp.zeros((16,), jnp.float32)

    def _zrow(i, carry):
        for j in range(D // 16):
            zbuf[i, pl.ds(j * 16, 16)] = zero
        return carry

    lax.fori_loop(0, CHUNK, _zrow, 0)
    for k in range(ROWS_PER_TILE // CHUNK):
        pltpu.sync_copy(zbuf, acc.at[pl.ds(s * ROWS_PER_TILE + k * CHUNK, CHUNK), :])
    plsc.subcore_barrier()

    pltpu.async_copy(x_hbm.at[sidx.at[0]], rows0, sem0)
    pltpu.async_copy(x_hbm.at[sidx.at[1]], rows1, sem1)

    def _pair(k, carry):
        j0 = k * 2
        j1 = j0 + 1
        pltpu.make_async_copy(x_hbm.at[sidx.at[j0]], rows0, sem0).wait()
        pltpu.sync_copy(rows0, acc.at[didx.at[j0]], add=True)
        pltpu.async_copy(x_hbm.at[sidx.at[j0 + 2]], rows0, sem0)
        pltpu.make_async_copy(x_hbm.at[sidx.at[j1]], rows1, sem1).wait()
        pltpu.sync_copy(rows1, acc.at[didx.at[j1]], add=True)
        pltpu.async_copy(x_hbm.at[sidx.at[j1 + 2]], rows1, sem1)
        return carry

    lax.fori_loop(0, N_CHUNKS // 2 - 1, _pair, 0)
    for j, (rows, sem) in ((N_CHUNKS - 2, (rows0, sem0)), (N_CHUNKS - 1, (rows1, sem1))):
        pltpu.make_async_copy(x_hbm.at[sidx.at[j]], rows, sem).wait()
        pltpu.sync_copy(rows, acc.at[didx.at[j]], add=True)
    plsc.subcore_barrier()

    pltpu.sync_copy(acc.at[pl.ds(s * ROWS_PER_TILE, ROWS_PER_TILE), :],
                    out_hbm.at[c, pl.ds(s * ROWS_PER_TILE, ROWS_PER_TILE), :])


_sc_segsum = pl.kernel(
    _sc_body,
    out_type=jax.ShapeDtypeStruct((N_CORES, N_NODES, D), jnp.float32),
    mesh=plsc.VectorSubcoreMesh(core_axis_name="c", subcore_axis_name="s"),
    scratch_types=[
        pltpu.VMEM((N_CHUNKS, CHUNK), jnp.int32),
        pltpu.VMEM((N_CHUNKS, CHUNK), jnp.int32),
        pltpu.VMEM((CHUNK, D), jnp.float32),
        pltpu.VMEM((CHUNK, D), jnp.float32),
        pltpu.VMEM((CHUNK, D), jnp.float32),
        pltpu.VMEM_SHARED((N_NODES, D), jnp.float32),
        pltpu.SemaphoreType.DMA,
        pltpu.SemaphoreType.DMA,
    ],
)


def _mm_body(p_ref, w_ref, b_ref, o_ref):
    h = p_ref[0] + p_ref[1]
    o_ref[...] = lax.dot_general(
        h, w_ref[...], (((1,), (1,)), ((), ())),
        preferred_element_type=jnp.float32) + b_ref[...]


ROW_BLK = 1000

_mm = pl.pallas_call(
    _mm_body,
    grid=(N_NODES // ROW_BLK,),
    in_specs=[
        pl.BlockSpec((N_CORES, ROW_BLK, D), lambda i: (0, i, 0)),
        pl.BlockSpec((D, D), lambda i: (0, 0)),
        pl.BlockSpec((1, D), lambda i: (0, 0)),
    ],
    out_specs=pl.BlockSpec((ROW_BLK, D), lambda i: (i, 0)),
    out_shape=jax.ShapeDtypeStruct((N_NODES, D), jnp.float32),
)


@jax.jit
def kernel(x, edge_index, W, b):
    src = edge_index[0].reshape(N_WORKERS, N_CHUNKS, CHUNK)
    dst = edge_index[1].reshape(N_WORKERS, N_CHUNKS, CHUNK)
    parts = _sc_segsum(x, src, dst)
    return _mm(parts, W, b.reshape(1, D))


# speedup vs baseline: 12.3814x; 12.3814x over previous
import jax
import jax.numpy as jnp
from jax import lax
from jax.experimental import pallas as pl
from jax.experimental.pallas import tpu as pltpu
from jax.experimental.pallas import tpu_sc as plsc

N_NODES = 10000
D = 128
N_CORES = 2
N_SUBCORES = 16
N_WORKERS = N_CORES * N_SUBCORES
EDGES_PER_TILE = 10000
CHUNK = 80
N_CHUNKS = EDGES_PER_TILE // CHUNK
ROWS_PER_TILE = 632
N_PAD = N_SUBCORES * ROWS_PER_TILE


def _mo(v):
    return pl.multiple_of(v, 8)


def _sc_body(x_hbm, src_hbm, dst_hbm, out_hbm,
             sidx, dch0, dch1, rows0, rows1, acc, sem0, sem1, semd0, semd1):
    c = lax.axis_index("c")
    s = lax.axis_index("s")
    wid = c * N_SUBCORES + s
    ebase = _mo(wid * EDGES_PER_TILE)

    pltpu.sync_copy(src_hbm.at[pl.ds(ebase, EDGES_PER_TILE)], sidx)

    zero = jnp.zeros((16,), jnp.float32)

    def _zrow(i, carry):
        for j in range(D // 16):
            rows0[i, pl.ds(j * 16, 16)] = zero
        return carry

    lax.fori_loop(0, CHUNK, _zrow, 0)
    abase = _mo(s * ROWS_PER_TILE)
    for k in range(ROWS_PER_TILE // CHUNK):
        pltpu.sync_copy(rows0, acc.at[pl.ds(_mo(abase + k * CHUNK), CHUNK), :])
    rem = ROWS_PER_TILE % CHUNK
    pltpu.sync_copy(rows0.at[pl.ds(0, rem), :],
                    acc.at[pl.ds(_mo(abase + ROWS_PER_TILE - rem), rem), :])
    plsc.subcore_barrier()

    def _start(j, rows, dch, semr, semd):
        off = _mo(j * CHUNK)
        pltpu.async_copy(x_hbm.at[sidx.at[pl.ds(off, CHUNK)]], rows, semr)
        pltpu.async_copy(dst_hbm.at[pl.ds(ebase + off, CHUNK)], dch, semd)

    def _finish(rows, dch, semr, semd):
        pltpu.make_async_copy(x_hbm.at[sidx.at[pl.ds(0, CHUNK)]], rows, semr).wait()
        pltpu.make_async_copy(dst_hbm.at[pl.ds(0, CHUNK)], dch, semd).wait()
        pltpu.sync_copy(rows, acc.at[dch], add=True)

    _start(0, rows0, dch0, sem0, semd0)
    _start(1, rows1, dch1, sem1, semd1)

    def _pair(k, carry):
        j0 = k * 2
        _finish(rows0, dch0, sem0, semd0)
        _start(j0 + 2, rows0, dch0, sem0, semd0)
        _finish(rows1, dch1, sem1, semd1)
        _start(j0 + 3, rows1, dch1, sem1, semd1)
        return carry

    lax.fori_loop(0, (N_CHUNKS - 3) // 2, _pair, 0)
    _finish(rows0, dch0, sem0, semd0)
    _start(N_CHUNKS - 1, rows0, dch0, sem0, semd0)
    _finish(rows1, dch1, sem1, semd1)
    _finish(rows0, dch0, sem0, semd0)
    plsc.subcore_barrier()

    pltpu.sync_copy(acc.at[pl.ds(abase, ROWS_PER_TILE), :],
                    out_hbm.at[c, pl.ds(abase, ROWS_PER_TILE), :])


_sc_segsum = pl.kernel(
    _sc_body,
    out_type=jax.ShapeDtypeStruct((N_CORES, N_PAD, D), jnp.float32),
    mesh=plsc.VectorSubcoreMesh(core_axis_name="c", subcore_axis_name="s"),
    scratch_types=[
        pltpu.VMEM((EDGES_PER_TILE,), jnp.int32),
        pltpu.VMEM((CHUNK,), jnp.int32),
        pltpu.VMEM((CHUNK,), jnp.int32),
        pltpu.VMEM((CHUNK, D), jnp.float32),
        pltpu.VMEM((CHUNK, D), jnp.float32),
        pltpu.VMEM_SHARED((N_PAD, D), jnp.float32),
        pltpu.SemaphoreType.DMA,
        pltpu.SemaphoreType.DMA,
        pltpu.SemaphoreType.DMA,
        pltpu.SemaphoreType.DMA,
    ],
)


def _mm_body(p_ref, w_ref, b_ref, o_ref):
    h = p_ref[0] + p_ref[1]
    o_ref[...] = lax.dot_general(
        h, w_ref[...], (((1,), (1,)), ((), ())),
        preferred_element_type=jnp.float32) + b_ref[...]


ROW_BLK = 1000

_mm = pl.pallas_call(
    _mm_body,
    grid=(N_NODES // ROW_BLK,),
    in_specs=[
        pl.BlockSpec((N_CORES, ROW_BLK, D), lambda i: (0, i, 0)),
        pl.BlockSpec((D, D), lambda i: (0, 0)),
        pl.BlockSpec((1, D), lambda i: (0, 0)),
    ],
    out_specs=pl.BlockSpec((ROW_BLK, D), lambda i: (i, 0)),
    out_shape=jax.ShapeDtypeStruct((N_NODES, D), jnp.float32),
)


@jax.jit
def kernel(x, edge_index, W, b):
    src = edge_index[0]
    dst = edge_index[1]
    parts = _sc_segsum(x, src, dst)
    return _mm(parts, W, b.reshape(1, D))
